# Initial kernel scaffold; baseline (speedup 1.0000x reference)
import jax
import jax.numpy as jnp
from jax.experimental import pallas as pl


def kernel(batch_points, batch_feats, batch_len):
    raise NotImplementedError("write your pallas kernel here")


# speedup vs baseline: 1.6964x; 1.6964x over previous
import jax
import jax.numpy as jnp
from jax.experimental import pallas as pl

_RADIUS2 = 0.25
_B = 2
_N = 4096
_TM = 512


def _adj_kernel(pi_ref, pjt_ref, out_ref):
    pi = pi_ref[0]
    pjt = pjt_ref[0]
    si = jnp.sum(pi * pi, axis=1, keepdims=True)
    sj = jnp.sum(pjt * pjt, axis=0, keepdims=True)
    m = jax.lax.dot_general(
        pi, pjt, (((1,), (0,)), ((), ())),
        preferred_element_type=jnp.float32)
    dist = (-2.0 * m + si) + sj
    out_ref[0] = dist <= _RADIUS2


def kernel(batch_points, batch_feats, batch_len):
    pts = batch_points.reshape(_B, _N, 3)
    fts = batch_feats.reshape(_B, _N, batch_feats.shape[-1])
    pts_t = jnp.swapaxes(pts, 1, 2)

    adj = pl.pallas_call(
        _adj_kernel,
        grid=(_B, _N // _TM),
        in_specs=[
            pl.BlockSpec((1, _TM, 3), lambda b, i: (b, i, 0)),
            pl.BlockSpec((1, 3, _N), lambda b, i: (b, 0, 0)),
        ],
        out_specs=pl.BlockSpec((1, _TM, _N), lambda b, i: (b, i, 0)),
        out_shape=jax.ShapeDtypeStruct((_B, _N, _N), jnp.bool_),
    )(pts, pts_t)
    return adj, pts, fts


# speedup vs baseline: 1.7034x; 1.0042x over previous
import jax
import jax.numpy as jnp
from jax.experimental import pallas as pl

_RADIUS2 = 0.25
_B = 2
_N = 4096
_TM = 512


def _adj_kernel(pi_ref, pjt_ref, out_ref):
    pi = pi_ref[0]
    pjt = pjt_ref[0]
    si = jnp.sum(pi * pi, axis=1, keepdims=True)
    sj = jnp.sum(pjt * pjt, axis=0, keepdims=True)
    m2 = jax.lax.dot_general(
        -2.0 * pi, pjt, (((1,), (0,)), ((), ())),
        preferred_element_type=jnp.float32)
    dist = (m2 + si) + sj
    out_ref[0] = dist <= _RADIUS2


def kernel(batch_points, batch_feats, batch_len):
    pts = batch_points.reshape(_B, _N, 3)
    fts = batch_feats.reshape(_B, _N, batch_feats.shape[-1])
    pts_t = jnp.swapaxes(pts, 1, 2)

    adj = pl.pallas_call(
        _adj_kernel,
        grid=(_B, _N // _TM),
        in_specs=[
            pl.BlockSpec((1, _TM, 3), lambda b, i: (b, i, 0)),
            pl.BlockSpec((1, 3, _N), lambda b, i: (b, 0, 0)),
        ],
        out_specs=pl.BlockSpec((1, _TM, _N), lambda b, i: (b, i, 0)),
        out_shape=jax.ShapeDtypeStruct((_B, _N, _N), jnp.bool_),
    )(pts, pts_t)
    return adj, pts, fts


# speedup vs baseline: 2.9509x; 1.7323x over previous
import jax
import jax.numpy as jnp
from jax.experimental import pallas as pl

_RADIUS2 = 0.25
_B = 2
_N = 4096
_TM = 512


def _adj_kernel(pi_ref, pjt_ref, out_ref):
    pi = pi_ref[0]
    pjt = pjt_ref[0]
    si = jnp.sum(pi * pi, axis=1, keepdims=True)
    sj = jnp.sum(pjt * pjt, axis=0, keepdims=True)
    m2 = jax.lax.dot_general(
        -2.0 * pi, pjt, (((1,), (0,)), ((), ())),
        preferred_element_type=jnp.float32)
    dist = (m2 + si) + sj
    out_ref[0] = (dist <= _RADIUS2).astype(jnp.int8)


def kernel(batch_points, batch_feats, batch_len):
    pts = batch_points.reshape(_B, _N, 3)
    fts = batch_feats.reshape(_B, _N, batch_feats.shape[-1])
    pts_t = jnp.swapaxes(pts, 1, 2)

    adj8 = pl.pallas_call(
        _adj_kernel,
        grid=(_B, _N // _TM),
        in_specs=[
            pl.BlockSpec((1, _TM, 3), lambda b, i: (b, i, 0)),
            pl.BlockSpec((1, 3, _N), lambda b, i: (b, 0, 0)),
        ],
        out_specs=pl.BlockSpec((1, _TM, _N), lambda b, i: (b, i, 0)),
        out_shape=jax.ShapeDtypeStruct((_B, _N, _N), jnp.int8),
    )(pts, pts_t)
    adj = adj8.view(jnp.bool_)
    return adj, pts, fts


# speedup vs baseline: 3.2610x; 1.1051x over previous
import jax
import jax.numpy as jnp
from jax.experimental import pallas as pl

_RADIUS2 = 0.25
_B = 2
_N = 4096
_TM = 512
_T = _N // _TM


def _adj_kernel(p_ref, pt_ref, out_ref):
    pt = pt_ref[0]
    sj_full = jnp.sum(pt * pt, axis=0, keepdims=True)
    for I in range(_T):
        pi = p_ref[0, I * _TM:(I + 1) * _TM, :]
        si = jnp.sum(pi * pi, axis=1, keepdims=True)
        npi = -2.0 * pi
        for J in range(I, _T):
            pjt = pt[:, J * _TM:(J + 1) * _TM]
            m2 = jax.lax.dot_general(
                npi, pjt, (((1,), (0,)), ((), ())),
                preferred_element_type=jnp.float32)
            sj = sj_full[:, J * _TM:(J + 1) * _TM]
            dist = (m2 + si) + sj
            v = (dist <= _RADIUS2).astype(jnp.int8)
            out_ref[0, I * _TM:(I + 1) * _TM, J * _TM:(J + 1) * _TM] = v
            if J != I:
                out_ref[0, J * _TM:(J + 1) * _TM, I * _TM:(I + 1) * _TM] = v.T


def kernel(batch_points, batch_feats, batch_len):
    pts = batch_points.reshape(_B, _N, 3)
    fts = batch_feats.reshape(_B, _N, batch_feats.shape[-1])
    pts_t = jnp.swapaxes(pts, 1, 2)

    adj8 = pl.pallas_call(
        _adj_kernel,
        grid=(_B,),
        in_specs=[
            pl.BlockSpec((1, _N, 3), lambda b: (b, 0, 0)),
            pl.BlockSpec((1, 3, _N), lambda b: (b, 0, 0)),
        ],
        out_specs=pl.BlockSpec((1, _N, _N), lambda b: (b, 0, 0)),
        out_shape=jax.ShapeDtypeStruct((_B, _N, _N), jnp.int8),
    )(pts, pts_t)
    adj = adj8.view(jnp.bool_)
    return adj, pts, fts
